# baseline (speedup 1.0000x reference)
import functools

import jax
import jax.numpy as jnp
from jax.experimental import pallas as pl
from jax.experimental.pallas import tpu as pltpu

_B = 64
_V = 2048
_D = 64
_A_BLK = 8
_N_ABLK = _B // _A_BLK
_N_COPIES = 2

_NEG = float("-inf")

_LANES = 128
_N_CHUNK = _V // _LANES


def _top2_chunks(chunks):
    hi = jnp.maximum(chunks[0], chunks[1])
    lo = jnp.minimum(chunks[0], chunks[1])
    for x in chunks[2:]:
        lo = jnp.maximum(lo, jnp.minimum(hi, x))
        hi = jnp.maximum(hi, x)
    m1 = jnp.max(hi, axis=1, keepdims=True)
    eq = hi == m1
    cnt = jnp.sum(jnp.where(eq, 1.0, 0.0), axis=1, keepdims=True)
    m2 = jnp.max(jnp.where(eq, lo, hi), axis=1, keepdims=True)
    m2 = jnp.where(cnt > 1.0, m1, m2)
    return m1, m2


def _rank_select(s0, s1):
    c = jnp.concatenate([s0, s1], axis=1)
    y = jnp.concatenate(
        [jnp.broadcast_to(s0[:, :, None], (_B, _B, _B)),
         jnp.broadcast_to(s1[:, :, None], (_B, _B, _B))], axis=2)
    x = c[:, None, :]
    gt = jnp.sum(jnp.where(y > x, 1.0, 0.0), axis=1)
    ge = jnp.sum(jnp.where(y >= x, 1.0, 0.0), axis=1)
    rank = jax.lax.broadcasted_iota(jnp.int32, (_B, 2 * _B), 0).astype(jnp.float32)
    cond = jnp.logical_and(gt <= rank, rank < ge)
    picked = jnp.max(jnp.where(cond[:, :_B], s0, _NEG), axis=1, keepdims=True)
    a_iota = jax.lax.broadcasted_iota(jnp.int32, (_B, _B), 1).astype(jnp.float32)
    first = jnp.min(jnp.where(cond[:, _B:], a_iota, float(_B)),
                    axis=1, keepdims=True)
    s1_m = jnp.where(a_iota == first, _NEG, s1)
    return picked, s1_m


def _loss_kernel(vis_hbm, lan_ref, tag_ref, out_ref, vbuf, vl0_s, vl1_s, sems):
    lan = lan_ref[:, 0, :]

    rows_per_copy = (_B // 2) // _N_COPIES
    copies = [
        pltpu.make_async_copy(
            vis_hbm.at[pl.ds(c * rows_per_copy, rows_per_copy)],
            vbuf.at[pl.ds(c * rows_per_copy, rows_per_copy)],
            sems.at[c])
        for c in range(_N_COPIES)
    ]
    for c in copies:
        c.start()

    half = _V // 2
    rows_per_blk = (_B // 2) // _N_ABLK
    waited = set()
    for k in range(_N_ABLK):
        c = (k * _N_COPIES) // _N_ABLK
        if c not in waited:
            copies[c].wait()
            waited.add(c)
        m1s, m2s = [], []
        for r in range(rows_per_blk):
            row = vbuf[k * rows_per_blk + r]
            se = jax.lax.dot_general(
                lan, row[:, :_D], (((1,), (1,)), ((), ())),
                preferred_element_type=jnp.float32)
            so = jax.lax.dot_general(
                lan, row[:, _D:], (((1,), (1,)), ((), ())),
                preferred_element_type=jnp.float32)
            for lohi in (0, 1):
                chunks = []
                for c in range(_N_CHUNK // 2):
                    base = lohi * half + c * _LANES
                    chunks.append(se[:, base:base + _LANES])
                    chunks.append(so[:, base:base + _LANES])
                m1, m2 = _top2_chunks(chunks)
                m1s.append(m1)
                m2s.append(m2)
        vl0_s[k] = jnp.concatenate(m1s, axis=1)
        vl1_s[k] = jnp.concatenate(m2s, axis=1)

    vl0 = vl0_s[...]
    vl1 = vl1_s[...]
    k_iota = jax.lax.broadcasted_iota(jnp.int32, vl0.shape, 0)
    b_iota = jax.lax.broadcasted_iota(jnp.int32, vl0.shape, 1)
    j_iota = jax.lax.broadcasted_iota(jnp.int32, vl0.shape, 2)
    diag = (k_iota * _A_BLK + j_iota) == b_iota
    picked_v = jnp.sum(jnp.where(diag, vl0, 0.0), axis=(0, 2))
    vl1_m = jnp.where(diag, _NEG, vl1)
    m_v = jnp.maximum(jnp.max(vl0, axis=(0, 2)), jnp.max(vl1_m, axis=(0, 2)))
    se_v = (jnp.sum(jnp.exp(vl0 - m_v[None, :, None]), axis=(0, 2))
            + jnp.sum(jnp.exp(vl1_m - m_v[None, :, None]), axis=(0, 2)))
    loss_v = jnp.mean(m_v + jnp.log(se_v) - picked_v)

    s0 = jax.lax.dot_general(lan, tag_ref[:, 0, :], (((1,), (1,)), ((), ())),
                             preferred_element_type=jnp.float32)
    s1 = jax.lax.dot_general(lan, tag_ref[:, 1, :], (((1,), (1,)), ((), ())),
                             preferred_element_type=jnp.float32)
    picked_t, s1_m = _rank_select(s0, s1)
    m_t = jnp.maximum(jnp.max(s0, axis=1, keepdims=True),
                      jnp.max(s1_m, axis=1, keepdims=True))
    se_t = (jnp.sum(jnp.exp(s0 - m_t), axis=1, keepdims=True)
            + jnp.sum(jnp.exp(s1_m - m_t), axis=1, keepdims=True))
    loss_t = jnp.mean(m_t + jnp.log(se_t) - picked_t)

    out_ref[...] = (loss_v + loss_t)[None, None]


@functools.partial(jax.jit, static_argnames=("interpret",))
def _run(vis_fs, lan_fs, tag_fs, interpret=False):
    vis2 = vis_fs.reshape(_B // 2, _V, 2 * _D)
    out = pl.pallas_call(
        _loss_kernel,
        in_specs=[
            pl.BlockSpec(memory_space=pl.ANY),
            pl.BlockSpec((_B, 1, _D), lambda: (0, 0, 0)),
            pl.BlockSpec((_B, 17, _D), lambda: (0, 0, 0)),
        ],
        out_specs=pl.BlockSpec((1, 1), lambda: (0, 0)),
        out_shape=jax.ShapeDtypeStruct((1, 1), jnp.float32),
        scratch_shapes=[
            pltpu.VMEM((_B // 2, _V, 2 * _D), jnp.float32),
            pltpu.VMEM((_N_ABLK, _B, _A_BLK), jnp.float32),
            pltpu.VMEM((_N_ABLK, _B, _A_BLK), jnp.float32),
            pltpu.SemaphoreType.DMA((_N_COPIES,)),
        ],
        interpret=interpret,
    )(vis2, lan_fs, tag_fs)
    return jnp.reshape(out, ())


def kernel(vis_fs, lan_fs, tag_fs):
    return _run(vis_fs, lan_fs, tag_fs)


# speedup vs baseline: 1.0217x; 1.0217x over previous
import functools

import jax
import jax.numpy as jnp
from jax.experimental import pallas as pl
from jax.experimental.pallas import tpu as pltpu

_B = 64
_V = 2048
_D = 64
_ROWS_BLK = 8
_A_BLK = 2 * _ROWS_BLK
_N_STEPS = _B // _A_BLK
_N_ABLK = _N_STEPS

_NEG = float("-inf")

_LANES = 128
_N_CHUNK = _V // _LANES


def _top2_chunks(chunks):
    hi = jnp.maximum(chunks[0], chunks[1])
    lo = jnp.minimum(chunks[0], chunks[1])
    for x in chunks[2:]:
        lo = jnp.maximum(lo, jnp.minimum(hi, x))
        hi = jnp.maximum(hi, x)
    m1 = jnp.max(hi, axis=1, keepdims=True)
    eq = hi == m1
    cnt = jnp.sum(jnp.where(eq, 1.0, 0.0), axis=1, keepdims=True)
    m2 = jnp.max(jnp.where(eq, lo, hi), axis=1, keepdims=True)
    m2 = jnp.where(cnt > 1.0, m1, m2)
    return m1, m2


def _rank_select(s0, s1):
    c = jnp.concatenate([s0, s1], axis=1)
    y = jnp.concatenate(
        [jnp.broadcast_to(s0[:, :, None], (_B, _B, _B)),
         jnp.broadcast_to(s1[:, :, None], (_B, _B, _B))], axis=2)
    x = c[:, None, :]
    gt = jnp.sum(jnp.where(y > x, 1.0, 0.0), axis=1)
    ge = jnp.sum(jnp.where(y >= x, 1.0, 0.0), axis=1)
    rank = jax.lax.broadcasted_iota(jnp.int32, (_B, 2 * _B), 0).astype(jnp.float32)
    cond = jnp.logical_and(gt <= rank, rank < ge)
    picked = jnp.max(jnp.where(cond[:, :_B], s0, _NEG), axis=1, keepdims=True)
    a_iota = jax.lax.broadcasted_iota(jnp.int32, (_B, _B), 1).astype(jnp.float32)
    first = jnp.min(jnp.where(cond[:, _B:], a_iota, float(_B)),
                    axis=1, keepdims=True)
    s1_m = jnp.where(a_iota == first, _NEG, s1)
    return picked, s1_m


def _loss_kernel(vis_ref, lan_ref, tag_ref, out_ref, vl0_s, vl1_s):
    i = pl.program_id(0)
    lan = lan_ref[:, 0, :]

    half = _V // 2
    m1s, m2s = [], []
    for r in range(_ROWS_BLK):
        row = vis_ref[r]
        se = jax.lax.dot_general(
            lan, row[:, :_D], (((1,), (1,)), ((), ())),
            preferred_element_type=jnp.float32)
        so = jax.lax.dot_general(
            lan, row[:, _D:], (((1,), (1,)), ((), ())),
            preferred_element_type=jnp.float32)
        for lohi in (0, 1):
            chunks = []
            for c in range(_N_CHUNK // 2):
                base = lohi * half + c * _LANES
                chunks.append(se[:, base:base + _LANES])
                chunks.append(so[:, base:base + _LANES])
            m1, m2 = _top2_chunks(chunks)
            m1s.append(m1)
            m2s.append(m2)
    vl0_s[pl.ds(i, 1)] = jnp.concatenate(m1s, axis=1)[None]
    vl1_s[pl.ds(i, 1)] = jnp.concatenate(m2s, axis=1)[None]

    @pl.when(i == _N_STEPS - 1)
    def _finalize():
        _final_losses(lan, tag_ref, out_ref, vl0_s, vl1_s)


def _final_losses(lan, tag_ref, out_ref, vl0_s, vl1_s):
    vl0 = vl0_s[...]
    vl1 = vl1_s[...]
    k_iota = jax.lax.broadcasted_iota(jnp.int32, vl0.shape, 0)
    b_iota = jax.lax.broadcasted_iota(jnp.int32, vl0.shape, 1)
    j_iota = jax.lax.broadcasted_iota(jnp.int32, vl0.shape, 2)
    diag = (k_iota * _A_BLK + j_iota) == b_iota
    picked_v = jnp.sum(jnp.where(diag, vl0, 0.0), axis=(0, 2))
    vl1_m = jnp.where(diag, _NEG, vl1)
    m_v = jnp.maximum(jnp.max(vl0, axis=(0, 2)), jnp.max(vl1_m, axis=(0, 2)))
    se_v = (jnp.sum(jnp.exp(vl0 - m_v[None, :, None]), axis=(0, 2))
            + jnp.sum(jnp.exp(vl1_m - m_v[None, :, None]), axis=(0, 2)))
    loss_v = jnp.mean(m_v + jnp.log(se_v) - picked_v)

    s0 = jax.lax.dot_general(lan, tag_ref[:, 0, :], (((1,), (1,)), ((), ())),
                             preferred_element_type=jnp.float32)
    s1 = jax.lax.dot_general(lan, tag_ref[:, 1, :], (((1,), (1,)), ((), ())),
                             preferred_element_type=jnp.float32)
    picked_t, s1_m = _rank_select(s0, s1)
    m_t = jnp.maximum(jnp.max(s0, axis=1, keepdims=True),
                      jnp.max(s1_m, axis=1, keepdims=True))
    se_t = (jnp.sum(jnp.exp(s0 - m_t), axis=1, keepdims=True)
            + jnp.sum(jnp.exp(s1_m - m_t), axis=1, keepdims=True))
    loss_t = jnp.mean(m_t + jnp.log(se_t) - picked_t)

    out_ref[...] = (loss_v + loss_t)[None, None]


@functools.partial(jax.jit, static_argnames=("interpret",))
def _run(vis_fs, lan_fs, tag_fs, interpret=False):
    vis2 = vis_fs.reshape(_B // 2, _V, 2 * _D)
    out = pl.pallas_call(
        _loss_kernel,
        grid=(_N_STEPS,),
        in_specs=[
            pl.BlockSpec((_ROWS_BLK, _V, 2 * _D), lambda i: (i, 0, 0)),
            pl.BlockSpec((_B, 1, _D), lambda i: (0, 0, 0)),
            pl.BlockSpec((_B, 17, _D), lambda i: (0, 0, 0)),
        ],
        out_specs=pl.BlockSpec((1, 1), lambda i: (0, 0)),
        out_shape=jax.ShapeDtypeStruct((1, 1), jnp.float32),
        scratch_shapes=[
            pltpu.VMEM((_N_ABLK, _B, _A_BLK), jnp.float32),
            pltpu.VMEM((_N_ABLK, _B, _A_BLK), jnp.float32),
        ],
        interpret=interpret,
    )(vis2, lan_fs, tag_fs)
    return jnp.reshape(out, ())


def kernel(vis_fs, lan_fs, tag_fs):
    return _run(vis_fs, lan_fs, tag_fs)


# speedup vs baseline: 1.5585x; 1.5253x over previous
import functools

import jax
import jax.numpy as jnp
from jax.experimental import pallas as pl
from jax.experimental.pallas import tpu as pltpu

_B = 64
_V = 2048
_D = 64
_A_BLK = 16
_N_SPLIT = 1
_N_ABLK = _B // _A_BLK
_N_STEPS = _N_ABLK // _N_SPLIT

_NEG = float("-inf")


_LANES = 128
_N_CHUNK = _V // _LANES


def _top2_lastaxis(s):
    c0 = s[:, 0:_LANES]
    c1 = s[:, _LANES:2 * _LANES]
    hi = jnp.maximum(c0, c1)
    lo = jnp.minimum(c0, c1)
    for c in range(2, _N_CHUNK):
        x = s[:, c * _LANES:(c + 1) * _LANES]
        lo = jnp.maximum(lo, jnp.minimum(hi, x))
        hi = jnp.maximum(hi, x)
    m1 = jnp.max(hi, axis=1, keepdims=True)
    eq = hi == m1
    cnt = jnp.sum(jnp.where(eq, 1.0, 0.0), axis=1, keepdims=True)
    m2 = jnp.max(jnp.where(eq, lo, hi), axis=1, keepdims=True)
    m2 = jnp.where(cnt > 1.0, m1, m2)
    return m1, m2


def _rank_select(s0, s1):
    c = jnp.concatenate([s0, s1], axis=1)
    y = jnp.concatenate(
        [jnp.broadcast_to(s0[:, :, None], (_B, _B, _B)),
         jnp.broadcast_to(s1[:, :, None], (_B, _B, _B))], axis=2)
    x = c[:, None, :]
    gt = jnp.sum(jnp.where(y > x, 1.0, 0.0), axis=1)
    ge = jnp.sum(jnp.where(y >= x, 1.0, 0.0), axis=1)
    rank = jax.lax.broadcasted_iota(jnp.int32, (_B, 2 * _B), 0).astype(jnp.float32)
    cond = jnp.logical_and(gt <= rank, rank < ge)
    picked = jnp.max(jnp.where(cond[:, :_B], s0, _NEG), axis=1, keepdims=True)
    a_iota = jax.lax.broadcasted_iota(jnp.int32, (_B, _B), 1).astype(jnp.float32)
    first = jnp.min(jnp.where(cond[:, _B:], a_iota, float(_B)),
                    axis=1, keepdims=True)
    s1_m = jnp.where(a_iota == first, _NEG, s1)
    return picked, s1_m


def _loss_kernel(*refs):
    vis_refs = refs[:_N_SPLIT]
    lan_ref, tag_ref, out_ref, vl0_s, vl1_s = refs[_N_SPLIT:]
    i = pl.program_id(0)
    lan = lan_ref[:, 0, :]

    for h, vref in enumerate(vis_refs):
        m1s, m2s = [], []
        for j in range(_A_BLK):
            s = jax.lax.dot_general(
                lan, vref[j],
                (((1,), (1,)), ((), ())),
                preferred_element_type=jnp.float32)
            m1, m2 = _top2_lastaxis(s)
            m1s.append(m1)
            m2s.append(m2)
        blk = i + h * _N_STEPS
        vl0_s[pl.ds(blk, 1)] = jnp.concatenate(m1s, axis=1)[None]
        vl1_s[pl.ds(blk, 1)] = jnp.concatenate(m2s, axis=1)[None]

    @pl.when(i == _N_STEPS - 1)
    def _finalize():
        vl0 = vl0_s[...]
        vl1 = vl1_s[...]
        s_iota = jax.lax.broadcasted_iota(jnp.int32, vl0.shape, 0)
        b_iota = jax.lax.broadcasted_iota(jnp.int32, vl0.shape, 1)
        j_iota = jax.lax.broadcasted_iota(jnp.int32, vl0.shape, 2)
        diag = (s_iota * _A_BLK + j_iota) == b_iota

        picked_v = jnp.sum(jnp.where(diag, vl0, 0.0), axis=(0, 2))
        vl1_m = jnp.where(diag, _NEG, vl1)
        m_v = jnp.maximum(jnp.max(vl0, axis=(0, 2)), jnp.max(vl1_m, axis=(0, 2)))
        se_v = (jnp.sum(jnp.exp(vl0 - m_v[None, :, None]), axis=(0, 2))
                + jnp.sum(jnp.exp(vl1_m - m_v[None, :, None]), axis=(0, 2)))
        loss_v = jnp.mean(m_v + jnp.log(se_v) - picked_v)

        s0 = jax.lax.dot_general(lan, tag_ref[:, 0, :], (((1,), (1,)), ((), ())),
                                 preferred_element_type=jnp.float32)
        s1 = jax.lax.dot_general(lan, tag_ref[:, 1, :], (((1,), (1,)), ((), ())),
                                 preferred_element_type=jnp.float32)
        picked_t, s1_m = _rank_select(s0, s1)
        m_t = jnp.maximum(jnp.max(s0, axis=1, keepdims=True),
                          jnp.max(s1_m, axis=1, keepdims=True))
        se_t = (jnp.sum(jnp.exp(s0 - m_t), axis=1, keepdims=True)
                + jnp.sum(jnp.exp(s1_m - m_t), axis=1, keepdims=True))
        loss_t = jnp.mean(m_t + jnp.log(se_t) - picked_t)

        out_ref[...] = (loss_v + loss_t)[None, None]


@functools.partial(jax.jit, static_argnames=("interpret",))
def _run(vis_fs, lan_fs, tag_fs, interpret=False):
    vis_specs = [
        pl.BlockSpec((_A_BLK, _V, _D),
                     functools.partial(lambda h, i: (h * _N_STEPS + i, 0, 0), h))
        for h in range(_N_SPLIT)
    ]
    out = pl.pallas_call(
        _loss_kernel,
        grid=(_N_STEPS,),
        in_specs=vis_specs + [
            pl.BlockSpec((_B, 1, _D), lambda i: (0, 0, 0)),
            pl.BlockSpec((_B, 17, _D), lambda i: (0, 0, 0)),
        ],
        out_specs=pl.BlockSpec((1, 1), lambda i: (0, 0)),
        out_shape=jax.ShapeDtypeStruct((1, 1), jnp.float32),
        scratch_shapes=[
            pltpu.VMEM((_N_ABLK, _B, _A_BLK), jnp.float32),
            pltpu.VMEM((_N_ABLK, _B, _A_BLK), jnp.float32),
        ],
        interpret=interpret,
    )(*([vis_fs] * _N_SPLIT), lan_fs, tag_fs)
    return jnp.reshape(out, ())


def kernel(vis_fs, lan_fs, tag_fs):
    return _run(vis_fs, lan_fs, tag_fs)


# speedup vs baseline: 7.6241x; 4.8920x over previous
import functools

import jax
import jax.numpy as jnp
from jax.experimental import pallas as pl
from jax.experimental.pallas import tpu as pltpu

_B = 64
_V = 2048
_D = 64
_A_BLK = 16
_N_SPLIT = 1
_N_ABLK = _B // _A_BLK
_N_STEPS = _N_ABLK // _N_SPLIT

_NEG = float("-inf")


_LANES = 128
_N_CHUNK = _V // _LANES


def _top2_lastaxis(s):
    c0 = s[:, 0:_LANES]
    c1 = s[:, _LANES:2 * _LANES]
    hi = jnp.maximum(c0, c1)
    lo = jnp.minimum(c0, c1)
    for c in range(2, _N_CHUNK):
        x = s[:, c * _LANES:(c + 1) * _LANES]
        lo = jnp.maximum(lo, jnp.minimum(hi, x))
        hi = jnp.maximum(hi, x)
    m1 = jnp.max(hi, axis=1, keepdims=True)
    eq = hi == m1
    cnt = jnp.sum(jnp.where(eq, 1.0, 0.0), axis=1, keepdims=True)
    m2 = jnp.max(jnp.where(eq, lo, hi), axis=1, keepdims=True)
    m2 = jnp.where(cnt > 1.0, m1, m2)
    return m1, m2


def _rank_select(s0, s1):
    c = jnp.concatenate([s0, s1], axis=1)
    y = jnp.concatenate(
        [jnp.broadcast_to(s0[:, :, None], (_B, _B, _B)),
         jnp.broadcast_to(s1[:, :, None], (_B, _B, _B))], axis=2)
    x = c[:, None, :]
    gt = jnp.sum(jnp.where(y > x, 1.0, 0.0), axis=1)
    ge = jnp.sum(jnp.where(y >= x, 1.0, 0.0), axis=1)
    rank = jax.lax.broadcasted_iota(jnp.int32, (_B, 2 * _B), 0).astype(jnp.float32)
    cond = jnp.logical_and(gt <= rank, rank < ge)
    picked = jnp.max(jnp.where(cond[:, :_B], s0, _NEG), axis=1, keepdims=True)
    a_iota = jax.lax.broadcasted_iota(jnp.int32, (_B, _B), 1).astype(jnp.float32)
    first = jnp.min(jnp.where(cond[:, _B:], a_iota, float(_B)),
                    axis=1, keepdims=True)
    s1_m = jnp.where(a_iota == first, _NEG, s1)
    return picked, s1_m


def _loss_kernel(vis_ref, lan_ref, tag_ref, out_ref, vl0_s, vl1_s):
    i = pl.program_id(0)
    lan = lan_ref[:, 0, :]

    m1s, m2s = [], []
    for j in range(_A_BLK):
        s = jax.lax.dot_general(
            lan, vis_ref[j],
            (((1,), (0,)), ((), ())),
            preferred_element_type=jnp.float32)
        m1, m2 = _top2_lastaxis(s)
        m1s.append(m1)
        m2s.append(m2)
    vl0_s[pl.ds(i, 1)] = jnp.concatenate(m1s, axis=1)[None]
    vl1_s[pl.ds(i, 1)] = jnp.concatenate(m2s, axis=1)[None]

    @pl.when(i == _N_STEPS - 1)
    def _finalize():
        vl0 = vl0_s[...]
        vl1 = vl1_s[...]
        s_iota = jax.lax.broadcasted_iota(jnp.int32, vl0.shape, 0)
        b_iota = jax.lax.broadcasted_iota(jnp.int32, vl0.shape, 1)
        j_iota = jax.lax.broadcasted_iota(jnp.int32, vl0.shape, 2)
        diag = (s_iota * _A_BLK + j_iota) == b_iota

        picked_v = jnp.sum(jnp.where(diag, vl0, 0.0), axis=(0, 2))
        vl1_m = jnp.where(diag, _NEG, vl1)
        m_v = jnp.maximum(jnp.max(vl0, axis=(0, 2)), jnp.max(vl1_m, axis=(0, 2)))
        se_v = (jnp.sum(jnp.exp(vl0 - m_v[None, :, None]), axis=(0, 2))
                + jnp.sum(jnp.exp(vl1_m - m_v[None, :, None]), axis=(0, 2)))
        loss_v = jnp.mean(m_v + jnp.log(se_v) - picked_v)

        s0 = jax.lax.dot_general(lan, tag_ref[0], (((1,), (1,)), ((), ())),
                                 preferred_element_type=jnp.float32)
        s1 = jax.lax.dot_general(lan, tag_ref[1], (((1,), (1,)), ((), ())),
                                 preferred_element_type=jnp.float32)
        picked_t, s1_m = _rank_select(s0, s1)
        m_t = jnp.maximum(jnp.max(s0, axis=1, keepdims=True),
                          jnp.max(s1_m, axis=1, keepdims=True))
        se_t = (jnp.sum(jnp.exp(s0 - m_t), axis=1, keepdims=True)
                + jnp.sum(jnp.exp(s1_m - m_t), axis=1, keepdims=True))
        loss_t = jnp.mean(m_t + jnp.log(se_t) - picked_t)

        out_ref[...] = (loss_v + loss_t)[None, None]


@functools.partial(jax.jit, static_argnames=("interpret",))
def _run(vis_fs, lan_fs, tag_fs, interpret=False):
    vis_t = jnp.transpose(vis_fs, (0, 2, 1))
    tag_t = jnp.transpose(tag_fs, (1, 0, 2))
    out = pl.pallas_call(
        _loss_kernel,
        grid=(_N_STEPS,),
        in_specs=[
            pl.BlockSpec((_A_BLK, _D, _V), lambda i: (i, 0, 0)),
            pl.BlockSpec((_B, 1, _D), lambda i: (0, 0, 0)),
            pl.BlockSpec((17, _B, _D), lambda i: (0, 0, 0)),
        ],
        out_specs=pl.BlockSpec((1, 1), lambda i: (0, 0)),
        out_shape=jax.ShapeDtypeStruct((1, 1), jnp.float32),
        scratch_shapes=[
            pltpu.VMEM((_N_ABLK, _B, _A_BLK), jnp.float32),
            pltpu.VMEM((_N_ABLK, _B, _A_BLK), jnp.float32),
        ],
        interpret=interpret,
    )(vis_t, lan_fs, tag_t)
    return jnp.reshape(out, ())


def kernel(vis_fs, lan_fs, tag_fs):
    return _run(vis_fs, lan_fs, tag_fs)
